# baseline (speedup 1.0000x reference)
import functools

import jax
import jax.numpy as jnp
from jax import lax
from jax.experimental import pallas as pl
from jax.experimental.pallas import tpu as pltpu
from jax.experimental.pallas import tpu_sc as plsc

USER = 5000
ITEM = 5000
N = USER + ITEM
E = 320000
D = 128
H = 4
DH = D // H

NW = 32
CE = 128
NP = N + 112
EP = 2560 * CE
CHUNKS_PER_W = (EP // CE) // NW
RPT = NP // 16

_MESH = plsc.VectorSubcoreMesh(core_axis_name="c", subcore_axis_name="s")

_GDN = lax.GatherDimensionNumbers(
    offset_dims=(), collapsed_slice_dims=(0,), start_index_map=(0,))


def _shuf(v, idx):
    return lax.gather(v, idx[:, None], _GDN, (1,),
                      mode=lax.GatherScatterMode.PROMISE_IN_BOUNDS)


def _head_sums(t, iot):
    t = [v + _shuf(v, iot ^ 8) for v in t]
    t = [v + _shuf(v, iot ^ 4) for v in t]
    m = jnp.where(iot < 4, t[0],
        jnp.where(iot < 8, t[1],
        jnp.where(iot < 12, t[2], t[3])))
    m = m + _shuf(m, iot ^ 2)
    m = m + _shuf(m, iot ^ 1)
    return m


NPAIR = CHUNKS_PER_W // 2


@functools.partial(
    pl.kernel,
    out_type=jax.ShapeDtypeStruct((2, NP, D), jnp.float32),
    mesh=_MESH,
    scratch_types=[
        pltpu.VMEM((CE,), jnp.int32),
        pltpu.VMEM((CE,), jnp.int32),
        pltpu.VMEM((CE // 16, 16), jnp.float32),
        pltpu.VMEM((CE, D), jnp.float32),
        pltpu.VMEM((CE,), jnp.int32),
        pltpu.VMEM((CE,), jnp.int32),
        pltpu.VMEM((CE // 16, 16), jnp.float32),
        pltpu.VMEM((CE, D), jnp.float32),
        pltpu.VMEM_SHARED((NP, D), jnp.float32),
        pltpu.SemaphoreType.DMA,
        pltpu.SemaphoreType.DMA,
    ],
)
def _sc_gcn(rows_hbm, cols_hbm, vals_hbm, table_hbm, zeros_hbm, out_hbm,
            rows_a, cols_a, vals_a, ebuf_a,
            rows_b, cols_b, vals_b, ebuf_b, acc_sp, sem_a, sem_b):
    cid = lax.axis_index("c")
    sid = lax.axis_index("s")
    wid = sid * 2 + cid
    pltpu.sync_copy(zeros_hbm.at[pl.ds(sid * RPT, RPT), :],
                    acc_sp.at[pl.ds(sid * RPT, RPT), :])
    plsc.subcore_barrier()

    def fetch(cidx, rows_buf, cols_buf, vals_buf, ebuf, sem):
        base = cidx * CE
        pltpu.sync_copy(rows_hbm.at[pl.ds(base, CE)], rows_buf)
        pltpu.sync_copy(cols_hbm.at[pl.ds(base, CE)], cols_buf)
        pltpu.sync_copy(vals_hbm.at[pl.ds(cidx * (CE // 16), CE // 16), :],
                        vals_buf)
        return pltpu.async_copy(table_hbm.at[cols_buf], ebuf, sem)

    def compute(rows_buf, vals_buf, ebuf):
        def grp_body(g, c):
            vals = vals_buf[g, :]
            for j in range(16):
                e = g * 16 + j
                sc = vals[j]
                for q in range(D // 16):
                    ebuf[e, pl.ds(q * 16, 16)] = \
                        ebuf[e, pl.ds(q * 16, 16)] * sc
            return c
        lax.fori_loop(0, CE // 16, grp_body, 0)
        pltpu.sync_copy(ebuf, acc_sp.at[rows_buf], add=True)

    cp0 = fetch(wid, rows_a, cols_a, vals_a, ebuf_a, sem_a)

    def body(i, carry):
        ca = wid + (2 * i) * NW
        cb = wid + (2 * i + 1) * NW
        cpb = fetch(cb, rows_b, cols_b, vals_b, ebuf_b, sem_b)
        pltpu.make_async_copy(table_hbm.at[cols_a], ebuf_a, sem_a).wait()
        compute(rows_a, vals_a, ebuf_a)
        cn = jnp.minimum(wid + (2 * i + 2) * NW, wid + (CHUNKS_PER_W - 2) * NW)
        fetch(cn, rows_a, cols_a, vals_a, ebuf_a, sem_a)
        cpb.wait()
        compute(rows_b, vals_b, ebuf_b)
        return carry

    lax.fori_loop(0, NPAIR, body, 0)
    pltpu.make_async_copy(table_hbm.at[cols_a], ebuf_a, sem_a).wait()
    plsc.subcore_barrier()
    pltpu.sync_copy(acc_sp.at[pl.ds(sid * RPT, RPT), :],
                    out_hbm.at[cid, pl.ds(sid * RPT, RPT), :])


CE1 = 64
CHUNKS_P1 = (EP // CE1) // NW
NPAIR1 = CHUNKS_P1 // 2


@functools.partial(
    pl.kernel,
    out_type=(jax.ShapeDtypeStruct((EP // 8, D), jnp.float32),
              jax.ShapeDtypeStruct((2, NP, D), jnp.float32)),
    mesh=_MESH,
    scratch_types=[
        pltpu.VMEM((CE1,), jnp.int32),
        pltpu.VMEM((CE1,), jnp.int32),
        pltpu.VMEM((CE1, D), jnp.float32),
        pltpu.VMEM((CE1, D), jnp.float32),
        pltpu.VMEM((CE1,), jnp.int32),
        pltpu.VMEM((CE1,), jnp.int32),
        pltpu.VMEM((CE1, D), jnp.float32),
        pltpu.VMEM((CE1, D), jnp.float32),
        pltpu.VMEM((CE1, D), jnp.float32),
        pltpu.VMEM((CE1 // 8, D), jnp.float32),
        pltpu.VMEM_SHARED((NP, D), jnp.float32),
        pltpu.SemaphoreType.DMA,
        pltpu.SemaphoreType.DMA,
        pltpu.SemaphoreType.DMA,
    ],
)
def _sc_gt_p1(rows_hbm, cols_hbm, q_hbm, k_hbm, f_hbm, zeros_hbm,
              ea_hbm, normp_hbm,
              rows_a, cols_a, qbuf_a, kbuf_a,
              rows_b, cols_b, qbuf_b, kbuf_b,
              fbuf, eabuf, norm_sp, sem_a, sem_b, sem_f):
    cid = lax.axis_index("c")
    sid = lax.axis_index("s")
    wid = sid * 2 + cid
    pltpu.sync_copy(zeros_hbm.at[pl.ds(sid * RPT, RPT), :],
                    norm_sp.at[pl.ds(sid * RPT, RPT), :])
    plsc.subcore_barrier()
    iot = lax.iota(jnp.int32, 16)

    def fetch(cidx, rows_buf, cols_buf, qbuf, kbuf, sem):
        base = cidx * CE1
        pltpu.sync_copy(rows_hbm.at[pl.ds(base, CE1)], rows_buf)
        pltpu.sync_copy(cols_hbm.at[pl.ds(base, CE1)], cols_buf)
        pltpu.async_copy(q_hbm.at[rows_buf], qbuf, sem)
        pltpu.async_copy(k_hbm.at[cols_buf], kbuf, sem)

    def wait_qk(qbuf, kbuf, sem):
        pltpu.make_async_copy(q_hbm.at[rows_a], qbuf, sem).wait()
        pltpu.make_async_copy(k_hbm.at[cols_a], kbuf, sem).wait()

    def compute(cidx, rows_buf, cols_buf, qbuf, kbuf):
        pltpu.async_copy(f_hbm.at[cols_buf], fbuf, sem_f).wait()

        def row_body(r, c):
            for j in range(8):
                e = r * 8 + j
                pr = [qbuf[e, pl.ds(q * 16, 16)] * kbuf[e, pl.ds(q * 16, 16)]
                      for q in range(D // 16)]
                att = _head_sums(
                    [pr[2 * h] + pr[2 * h + 1] for h in range(H)], iot)
                att = jnp.clip(att, -10.0, 10.0) + fbuf[e, pl.ds(0, 16)]
                ex = jnp.exp(att)
                fbuf[e, pl.ds(0, 16)] = ex
                eabuf[r, pl.ds(j * 16, 16)] = ex
            return c

        lax.fori_loop(0, CE1 // 8, row_body, 0)
        pltpu.sync_copy(eabuf,
                        ea_hbm.at[pl.ds(cidx * (CE1 // 8), CE1 // 8), :])
        pltpu.sync_copy(fbuf, norm_sp.at[rows_buf], add=True)

    fetch(wid, rows_a, cols_a, qbuf_a, kbuf_a, sem_a)

    def body(i, carry):
        ca = wid + (2 * i) * NW
        cb = wid + (2 * i + 1) * NW
        fetch(cb, rows_b, cols_b, qbuf_b, kbuf_b, sem_b)
        wait_qk(qbuf_a, kbuf_a, sem_a)
        compute(ca, rows_a, cols_a, qbuf_a, kbuf_a)
        cn = jnp.minimum(wid + (2 * i + 2) * NW, wid + (CHUNKS_P1 - 2) * NW)
        fetch(cn, rows_a, cols_a, qbuf_a, kbuf_a, sem_a)
        wait_qk(qbuf_b, kbuf_b, sem_b)
        compute(cb, rows_b, cols_b, qbuf_b, kbuf_b)
        return carry

    lax.fori_loop(0, NPAIR1, body, 0)
    wait_qk(qbuf_a, kbuf_a, sem_a)
    plsc.subcore_barrier()
    pltpu.sync_copy(norm_sp.at[pl.ds(sid * RPT, RPT), :],
                    normp_hbm.at[cid, pl.ds(sid * RPT, RPT), :])


@functools.partial(
    pl.kernel,
    out_type=jax.ShapeDtypeStruct((2, NP, D), jnp.float32),
    mesh=_MESH,
    scratch_types=[
        pltpu.VMEM((CE,), jnp.int32),
        pltpu.VMEM((CE,), jnp.int32),
        pltpu.VMEM((CE, D), jnp.float32),
        pltpu.VMEM((CE // 8, D), jnp.float32),
        pltpu.VMEM((CE,), jnp.int32),
        pltpu.VMEM((CE,), jnp.int32),
        pltpu.VMEM((CE, D), jnp.float32),
        pltpu.VMEM((CE // 8, D), jnp.float32),
        pltpu.VMEM_SHARED((NP, D), jnp.float32),
        pltpu.SemaphoreType.DMA,
        pltpu.SemaphoreType.DMA,
    ],
)
def _sc_gt_p2(rows_hbm, cols_hbm, v_hbm, ea_hbm, zeros_hbm, out_hbm,
              rows_a, cols_a, vbuf_a, abuf_a,
              rows_b, cols_b, vbuf_b, abuf_b, acc_sp, sem_a, sem_b):
    cid = lax.axis_index("c")
    sid = lax.axis_index("s")
    wid = sid * 2 + cid
    pltpu.sync_copy(zeros_hbm.at[pl.ds(sid * RPT, RPT), :],
                    acc_sp.at[pl.ds(sid * RPT, RPT), :])
    plsc.subcore_barrier()

    def fetch(cidx, rows_buf, cols_buf, vbuf, abuf, sem):
        base = cidx * CE
        pltpu.sync_copy(rows_hbm.at[pl.ds(base, CE)], rows_buf)
        pltpu.sync_copy(cols_hbm.at[pl.ds(base, CE)], cols_buf)
        pltpu.sync_copy(ea_hbm.at[pl.ds(cidx * (CE // 8), CE // 8), :], abuf)
        pltpu.async_copy(v_hbm.at[cols_buf], vbuf, sem)

    def compute(rows_buf, vbuf, abuf):
        def row_body(r, c):
            for j in range(8):
                e = r * 8 + j
                a = abuf[r, pl.ds(j * 16, 16)]
                for h in range(H):
                    sh = a[4 * h]
                    vbuf[e, pl.ds(2 * h * 16, 16)] = \
                        vbuf[e, pl.ds(2 * h * 16, 16)] * sh
                    vbuf[e, pl.ds((2 * h + 1) * 16, 16)] = \
                        vbuf[e, pl.ds((2 * h + 1) * 16, 16)] * sh
            return c

        lax.fori_loop(0, CE // 8, row_body, 0)
        pltpu.sync_copy(vbuf, acc_sp.at[rows_buf], add=True)

    fetch(wid, rows_a, cols_a, vbuf_a, abuf_a, sem_a)

    def body(i, carry):
        cb = wid + (2 * i + 1) * NW
        fetch(cb, rows_b, cols_b, vbuf_b, abuf_b, sem_b)
        pltpu.make_async_copy(v_hbm.at[cols_a], vbuf_a, sem_a).wait()
        compute(rows_a, vbuf_a, abuf_a)
        cn = jnp.minimum(wid + (2 * i + 2) * NW, wid + (CHUNKS_PER_W - 2) * NW)
        fetch(cn, rows_a, cols_a, vbuf_a, abuf_a, sem_a)
        pltpu.make_async_copy(v_hbm.at[cols_b], vbuf_b, sem_b).wait()
        compute(rows_b, vbuf_b, abuf_b)
        return carry

    lax.fori_loop(0, NPAIR, body, 0)
    pltpu.make_async_copy(v_hbm.at[cols_a], vbuf_a, sem_a).wait()
    plsc.subcore_barrier()
    pltpu.sync_copy(acc_sp.at[pl.ds(sid * RPT, RPT), :],
                    out_hbm.at[cid, pl.ds(sid * RPT, RPT), :])


_BLK = NP // 8


def _norm_div(t0_ref, t1_ref, n0_ref, n1_ref):
    n = n0_ref[...] + n1_ref[...] + 1e-8
    n = jnp.repeat(n, DH // 4, axis=1)
    return (t0_ref[...] + t1_ref[...]) / n


def _tc_qkv(a0, a1, n0, n1, w):
    def body(a0_ref, a1_ref, n0_ref, n1_ref, w_ref, comb_ref, qkv_ref):
        a = _norm_div(a0_ref, a1_ref, n0_ref, n1_ref)
        comb_ref[...] = a
        qkv_ref[...] = jnp.dot(a, w_ref[...],
                               preferred_element_type=jnp.float32)
    return pl.pallas_call(
        body,
        grid=(NP // _BLK,),
        in_specs=[pl.BlockSpec((_BLK, D), lambda i: (i, 0)),
                  pl.BlockSpec((_BLK, D), lambda i: (i, 0)),
                  pl.BlockSpec((_BLK, 16), lambda i: (i, 0)),
                  pl.BlockSpec((_BLK, 16), lambda i: (i, 0)),
                  pl.BlockSpec((D, 3 * D), lambda i: (0, 0))],
        out_specs=[pl.BlockSpec((_BLK, D), lambda i: (i, 0)),
                   pl.BlockSpec((_BLK, 3 * D), lambda i: (i, 0))],
        out_shape=[jax.ShapeDtypeStruct((NP, D), jnp.float32),
                   jax.ShapeDtypeStruct((NP, 3 * D), jnp.float32)],
    )(a0, a1, n0, n1, w)


def _tc_qkv0(a0, a1, w):
    def body(a0_ref, a1_ref, w_ref, comb_ref, qkv_ref):
        a = a0_ref[...] + a1_ref[...]
        comb_ref[...] = a
        qkv_ref[...] = jnp.dot(a, w_ref[...],
                               preferred_element_type=jnp.float32)
    return pl.pallas_call(
        body,
        grid=(NP // _BLK,),
        in_specs=[pl.BlockSpec((_BLK, D), lambda i: (i, 0)),
                  pl.BlockSpec((_BLK, D), lambda i: (i, 0)),
                  pl.BlockSpec((D, 3 * D), lambda i: (0, 0))],
        out_specs=[pl.BlockSpec((_BLK, D), lambda i: (i, 0)),
                   pl.BlockSpec((_BLK, 3 * D), lambda i: (i, 0))],
        out_shape=[jax.ShapeDtypeStruct((NP, D), jnp.float32),
                   jax.ShapeDtypeStruct((NP, 3 * D), jnp.float32)],
    )(a0, a1, w)


def _tc_sum2(a0, a1):
    def body(a0_ref, a1_ref, o_ref):
        o_ref[...] = a0_ref[...] + a1_ref[...]
    return pl.pallas_call(
        body,
        grid=(NP // _BLK,),
        in_specs=[pl.BlockSpec((_BLK, D), lambda i: (i, 0))] * 2,
        out_specs=pl.BlockSpec((_BLK, D), lambda i: (i, 0)),
        out_shape=jax.ShapeDtypeStruct((NP, D), jnp.float32),
    )(a0, a1)


def _tc_final(e0, e1, e2, e3, t0, t1, n0, n1):
    def body(e0_ref, e1_ref, e2_ref, e3_ref, t0_ref, t1_ref, n0_ref, n1_ref,
             o_ref):
        e4 = _norm_div(t0_ref, t1_ref, n0_ref, n1_ref)
        o_ref[...] = (e0_ref[...] + e1_ref[...] + e2_ref[...]
                      + e3_ref[...] + e4)
    return pl.pallas_call(
        body,
        grid=(NP // _BLK,),
        in_specs=[pl.BlockSpec((_BLK, D), lambda i: (i, 0))] * 6
                 + [pl.BlockSpec((_BLK, 16), lambda i: (i, 0))] * 2,
        out_specs=pl.BlockSpec((_BLK, D), lambda i: (i, 0)),
        out_shape=jax.ShapeDtypeStruct((NP, D), jnp.float32),
    )(e0, e1, e2, e3, t0, t1, n0, n1)


def kernel(enc_edge_index, enc_values, dec_edge_index, uEmbeds, iEmbeds,
           qTrans0, kTrans0, vTrans0, filter0,
           qTrans1, kTrans1, vTrans1, filter1):
    f32 = jnp.float32
    e0 = jnp.concatenate(
        [uEmbeds, iEmbeds, jnp.zeros((NP - N, D), f32)], axis=0)
    zeros_nd = jnp.zeros((NP, D), f32)
    zeros_n16 = jnp.zeros((NP, 16), f32)

    pad = EP - E
    enc_rows = jnp.concatenate(
        [enc_edge_index[0], jnp.full((pad,), N, jnp.int32)])
    enc_cols = jnp.concatenate(
        [enc_edge_index[1], jnp.full((pad,), N, jnp.int32)])
    enc_vals = jnp.concatenate(
        [enc_values, jnp.zeros((pad,), f32)]).reshape(EP // 16, 16)
    dec_rows = jnp.concatenate(
        [dec_edge_index[0], jnp.full((pad,), N, jnp.int32)])
    dec_cols = jnp.concatenate(
        [dec_edge_index[1], jnp.full((pad,), N, jnp.int32)])

    f0p = jnp.pad(jnp.repeat(filter0, 4, axis=1), ((0, NP - N), (0, D - 16)))
    f1p = jnp.pad(jnp.repeat(filter1, 4, axis=1), ((0, NP - N), (0, D - 16)))
    w0 = jnp.concatenate([qTrans0, kTrans0, vTrans0], axis=1)
    w1 = jnp.concatenate([qTrans1, kTrans1, vTrans1], axis=1)

    g1 = _sc_gcn(enc_rows, enc_cols, enc_vals, e0, zeros_nd)
    e1 = _tc_sum2(g1[0], g1[1])
    g2 = _sc_gcn(enc_rows, enc_cols, enc_vals, e1, zeros_nd)

    e2, qkv1 = _tc_qkv0(g2[0], g2[1], w0)
    ea1, np1 = _sc_gt_p1(dec_rows, dec_cols, qkv1[:, :D], qkv1[:, D:2 * D],
                         f0p, zeros_nd)
    t1 = _sc_gt_p2(dec_rows, dec_cols, qkv1[:, 2 * D:], ea1, zeros_nd)

    e3, qkv2 = _tc_qkv(t1[0], t1[1], np1[0][:, :16], np1[1][:, :16], w1)
    ea2, np2 = _sc_gt_p1(dec_rows, dec_cols, qkv2[:, :D], qkv2[:, D:2 * D],
                         f1p, zeros_nd)
    t2 = _sc_gt_p2(dec_rows, dec_cols, qkv2[:, 2 * D:], ea2, zeros_nd)

    out = _tc_final(e0, e1, e2, e3, t2[0], t2[1],
                    np2[0][:, :16], np2[1][:, :16])
    return (out[:USER], out[USER:N])


# speedup vs baseline: 1.0848x; 1.0848x over previous
import functools

import jax
import jax.numpy as jnp
from jax import lax
from jax.experimental import pallas as pl
from jax.experimental.pallas import tpu as pltpu
from jax.experimental.pallas import tpu_sc as plsc

USER = 5000
ITEM = 5000
N = USER + ITEM
E = 320000
D = 128
H = 4
DH = D // H

NW = 32
CE = 128
NP = N + 112
EP = 2560 * CE
CHUNKS_PER_W = (EP // CE) // NW
RPT = NP // 16

_MESH = plsc.VectorSubcoreMesh(core_axis_name="c", subcore_axis_name="s")

_GDN = lax.GatherDimensionNumbers(
    offset_dims=(), collapsed_slice_dims=(0,), start_index_map=(0,))


def _shuf(v, idx):
    return lax.gather(v, idx[:, None], _GDN, (1,),
                      mode=lax.GatherScatterMode.PROMISE_IN_BOUNDS)


def _head_sums(t, iot):
    t = [v + _shuf(v, iot ^ 8) for v in t]
    t = [v + _shuf(v, iot ^ 4) for v in t]
    m = jnp.where(iot < 4, t[0],
        jnp.where(iot < 8, t[1],
        jnp.where(iot < 12, t[2], t[3])))
    m = m + _shuf(m, iot ^ 2)
    m = m + _shuf(m, iot ^ 1)
    return m


NPAIR = CHUNKS_PER_W // 2


@functools.partial(
    pl.kernel,
    out_type=jax.ShapeDtypeStruct((2, NP, D), jnp.float32),
    mesh=_MESH,
    scratch_types=[
        pltpu.VMEM((CE,), jnp.int32),
        pltpu.VMEM((CE,), jnp.int32),
        pltpu.VMEM((CE // 16, 16), jnp.float32),
        pltpu.VMEM((CE, D), jnp.float32),
        pltpu.VMEM((CE,), jnp.int32),
        pltpu.VMEM((CE,), jnp.int32),
        pltpu.VMEM((CE // 16, 16), jnp.float32),
        pltpu.VMEM((CE, D), jnp.float32),
        pltpu.VMEM_SHARED((NP, D), jnp.float32),
        pltpu.SemaphoreType.DMA,
        pltpu.SemaphoreType.DMA,
    ],
)
def _sc_gcn(rows_hbm, cols_hbm, vals_hbm, table_hbm, zeros_hbm, out_hbm,
            rows_a, cols_a, vals_a, ebuf_a,
            rows_b, cols_b, vals_b, ebuf_b, acc_sp, sem_a, sem_b):
    cid = lax.axis_index("c")
    sid = lax.axis_index("s")
    wid = sid * 2 + cid
    pltpu.sync_copy(zeros_hbm.at[pl.ds(sid * RPT, RPT), :],
                    acc_sp.at[pl.ds(sid * RPT, RPT), :])
    plsc.subcore_barrier()

    def fetch(cidx, rows_buf, cols_buf, vals_buf, ebuf, sem):
        base = cidx * CE
        pltpu.sync_copy(rows_hbm.at[pl.ds(base, CE)], rows_buf)
        pltpu.sync_copy(cols_hbm.at[pl.ds(base, CE)], cols_buf)
        pltpu.sync_copy(vals_hbm.at[pl.ds(cidx * (CE // 16), CE // 16), :],
                        vals_buf)
        return pltpu.async_copy(table_hbm.at[cols_buf], ebuf, sem)

    def compute(rows_buf, vals_buf, ebuf):
        @plsc.parallel_loop(0, CE // 16)
        def grp_body(g):
            vals = vals_buf[g, :]
            for j in range(16):
                e = g * 16 + j
                sc = vals[j]
                for q in range(D // 16):
                    ebuf[e, pl.ds(q * 16, 16)] = \
                        ebuf[e, pl.ds(q * 16, 16)] * sc
        pltpu.sync_copy(ebuf, acc_sp.at[rows_buf], add=True)

    cp0 = fetch(wid, rows_a, cols_a, vals_a, ebuf_a, sem_a)

    def body(i, carry):
        ca = wid + (2 * i) * NW
        cb = wid + (2 * i + 1) * NW
        cpb = fetch(cb, rows_b, cols_b, vals_b, ebuf_b, sem_b)
        pltpu.make_async_copy(table_hbm.at[cols_a], ebuf_a, sem_a).wait()
        compute(rows_a, vals_a, ebuf_a)
        cn = jnp.minimum(wid + (2 * i + 2) * NW, wid + (CHUNKS_PER_W - 2) * NW)
        fetch(cn, rows_a, cols_a, vals_a, ebuf_a, sem_a)
        cpb.wait()
        compute(rows_b, vals_b, ebuf_b)
        return carry

    lax.fori_loop(0, NPAIR, body, 0)
    pltpu.make_async_copy(table_hbm.at[cols_a], ebuf_a, sem_a).wait()
    plsc.subcore_barrier()
    pltpu.sync_copy(acc_sp.at[pl.ds(sid * RPT, RPT), :],
                    out_hbm.at[cid, pl.ds(sid * RPT, RPT), :])


CE1 = 64
CHUNKS_P1 = (EP // CE1) // NW
NPAIR1 = CHUNKS_P1 // 2


@functools.partial(
    pl.kernel,
    out_type=(jax.ShapeDtypeStruct((EP // 8, D), jnp.float32),
              jax.ShapeDtypeStruct((2, NP, D), jnp.float32)),
    mesh=_MESH,
    scratch_types=[
        pltpu.VMEM((CE1,), jnp.int32),
        pltpu.VMEM((CE1,), jnp.int32),
        pltpu.VMEM((CE1, D), jnp.float32),
        pltpu.VMEM((CE1, D), jnp.float32),
        pltpu.VMEM((CE1,), jnp.int32),
        pltpu.VMEM((CE1,), jnp.int32),
        pltpu.VMEM((CE1, D), jnp.float32),
        pltpu.VMEM((CE1, D), jnp.float32),
        pltpu.VMEM((CE1, D), jnp.float32),
        pltpu.VMEM((CE1 // 8, D), jnp.float32),
        pltpu.VMEM_SHARED((NP, D), jnp.float32),
        pltpu.SemaphoreType.DMA,
        pltpu.SemaphoreType.DMA,
        pltpu.SemaphoreType.DMA,
    ],
)
def _sc_gt_p1(rows_hbm, cols_hbm, q_hbm, k_hbm, f_hbm, zeros_hbm,
              ea_hbm, normp_hbm,
              rows_a, cols_a, qbuf_a, kbuf_a,
              rows_b, cols_b, qbuf_b, kbuf_b,
              fbuf, eabuf, norm_sp, sem_a, sem_b, sem_f):
    cid = lax.axis_index("c")
    sid = lax.axis_index("s")
    wid = sid * 2 + cid
    pltpu.sync_copy(zeros_hbm.at[pl.ds(sid * RPT, RPT), :],
                    norm_sp.at[pl.ds(sid * RPT, RPT), :])
    plsc.subcore_barrier()
    iot = lax.iota(jnp.int32, 16)

    def fetch(cidx, rows_buf, cols_buf, qbuf, kbuf, sem):
        base = cidx * CE1
        pltpu.sync_copy(rows_hbm.at[pl.ds(base, CE1)], rows_buf)
        pltpu.sync_copy(cols_hbm.at[pl.ds(base, CE1)], cols_buf)
        pltpu.async_copy(q_hbm.at[rows_buf], qbuf, sem)
        pltpu.async_copy(k_hbm.at[cols_buf], kbuf, sem)

    def wait_qk(qbuf, kbuf, sem):
        pltpu.make_async_copy(q_hbm.at[rows_a], qbuf, sem).wait()
        pltpu.make_async_copy(k_hbm.at[cols_a], kbuf, sem).wait()

    def compute(cidx, rows_buf, cols_buf, qbuf, kbuf):
        pltpu.async_copy(f_hbm.at[cols_buf], fbuf, sem_f).wait()

        @plsc.parallel_loop(0, CE1 // 8)
        def row_body(r):
            for j in range(8):
                e = r * 8 + j
                pr = [qbuf[e, pl.ds(q * 16, 16)] * kbuf[e, pl.ds(q * 16, 16)]
                      for q in range(D // 16)]
                att = _head_sums(
                    [pr[2 * h] + pr[2 * h + 1] for h in range(H)], iot)
                att = jnp.clip(att, -10.0, 10.0) + fbuf[e, pl.ds(0, 16)]
                ex = jnp.exp(att)
                fbuf[e, pl.ds(0, 16)] = ex
                eabuf[r, pl.ds(j * 16, 16)] = ex
        pltpu.sync_copy(eabuf,
                        ea_hbm.at[pl.ds(cidx * (CE1 // 8), CE1 // 8), :])
        pltpu.sync_copy(fbuf, norm_sp.at[rows_buf], add=True)

    fetch(wid, rows_a, cols_a, qbuf_a, kbuf_a, sem_a)

    def body(i, carry):
        ca = wid + (2 * i) * NW
        cb = wid + (2 * i + 1) * NW
        fetch(cb, rows_b, cols_b, qbuf_b, kbuf_b, sem_b)
        wait_qk(qbuf_a, kbuf_a, sem_a)
        compute(ca, rows_a, cols_a, qbuf_a, kbuf_a)
        cn = jnp.minimum(wid + (2 * i + 2) * NW, wid + (CHUNKS_P1 - 2) * NW)
        fetch(cn, rows_a, cols_a, qbuf_a, kbuf_a, sem_a)
        wait_qk(qbuf_b, kbuf_b, sem_b)
        compute(cb, rows_b, cols_b, qbuf_b, kbuf_b)
        return carry

    lax.fori_loop(0, NPAIR1, body, 0)
    wait_qk(qbuf_a, kbuf_a, sem_a)
    plsc.subcore_barrier()
    pltpu.sync_copy(norm_sp.at[pl.ds(sid * RPT, RPT), :],
                    normp_hbm.at[cid, pl.ds(sid * RPT, RPT), :])


@functools.partial(
    pl.kernel,
    out_type=jax.ShapeDtypeStruct((2, NP, D), jnp.float32),
    mesh=_MESH,
    scratch_types=[
        pltpu.VMEM((CE,), jnp.int32),
        pltpu.VMEM((CE,), jnp.int32),
        pltpu.VMEM((CE, D), jnp.float32),
        pltpu.VMEM((CE // 8, D), jnp.float32),
        pltpu.VMEM((CE,), jnp.int32),
        pltpu.VMEM((CE,), jnp.int32),
        pltpu.VMEM((CE, D), jnp.float32),
        pltpu.VMEM((CE // 8, D), jnp.float32),
        pltpu.VMEM_SHARED((NP, D), jnp.float32),
        pltpu.SemaphoreType.DMA,
        pltpu.SemaphoreType.DMA,
    ],
)
def _sc_gt_p2(rows_hbm, cols_hbm, v_hbm, ea_hbm, zeros_hbm, out_hbm,
              rows_a, cols_a, vbuf_a, abuf_a,
              rows_b, cols_b, vbuf_b, abuf_b, acc_sp, sem_a, sem_b):
    cid = lax.axis_index("c")
    sid = lax.axis_index("s")
    wid = sid * 2 + cid
    pltpu.sync_copy(zeros_hbm.at[pl.ds(sid * RPT, RPT), :],
                    acc_sp.at[pl.ds(sid * RPT, RPT), :])
    plsc.subcore_barrier()

    def fetch(cidx, rows_buf, cols_buf, vbuf, abuf, sem):
        base = cidx * CE
        pltpu.sync_copy(rows_hbm.at[pl.ds(base, CE)], rows_buf)
        pltpu.sync_copy(cols_hbm.at[pl.ds(base, CE)], cols_buf)
        pltpu.sync_copy(ea_hbm.at[pl.ds(cidx * (CE // 8), CE // 8), :], abuf)
        pltpu.async_copy(v_hbm.at[cols_buf], vbuf, sem)

    def compute(rows_buf, vbuf, abuf):
        @plsc.parallel_loop(0, CE // 8)
        def row_body(r):
            for j in range(8):
                e = r * 8 + j
                a = abuf[r, pl.ds(j * 16, 16)]
                for h in range(H):
                    sh = a[4 * h]
                    vbuf[e, pl.ds(2 * h * 16, 16)] = \
                        vbuf[e, pl.ds(2 * h * 16, 16)] * sh
                    vbuf[e, pl.ds((2 * h + 1) * 16, 16)] = \
                        vbuf[e, pl.ds((2 * h + 1) * 16, 16)] * sh
        pltpu.sync_copy(vbuf, acc_sp.at[rows_buf], add=True)

    fetch(wid, rows_a, cols_a, vbuf_a, abuf_a, sem_a)

    def body(i, carry):
        cb = wid + (2 * i + 1) * NW
        fetch(cb, rows_b, cols_b, vbuf_b, abuf_b, sem_b)
        pltpu.make_async_copy(v_hbm.at[cols_a], vbuf_a, sem_a).wait()
        compute(rows_a, vbuf_a, abuf_a)
        cn = jnp.minimum(wid + (2 * i + 2) * NW, wid + (CHUNKS_PER_W - 2) * NW)
        fetch(cn, rows_a, cols_a, vbuf_a, abuf_a, sem_a)
        pltpu.make_async_copy(v_hbm.at[cols_b], vbuf_b, sem_b).wait()
        compute(rows_b, vbuf_b, abuf_b)
        return carry

    lax.fori_loop(0, NPAIR, body, 0)
    pltpu.make_async_copy(v_hbm.at[cols_a], vbuf_a, sem_a).wait()
    plsc.subcore_barrier()
    pltpu.sync_copy(acc_sp.at[pl.ds(sid * RPT, RPT), :],
                    out_hbm.at[cid, pl.ds(sid * RPT, RPT), :])


_BLK = NP // 8


def _norm_div(t0_ref, t1_ref, n0_ref, n1_ref):
    n = n0_ref[...] + n1_ref[...] + 1e-8
    n = jnp.repeat(n, DH // 4, axis=1)
    return (t0_ref[...] + t1_ref[...]) / n


def _tc_qkv(a0, a1, n0, n1, w):
    def body(a0_ref, a1_ref, n0_ref, n1_ref, w_ref, comb_ref, qkv_ref):
        a = _norm_div(a0_ref, a1_ref, n0_ref, n1_ref)
        comb_ref[...] = a
        qkv_ref[...] = jnp.dot(a, w_ref[...],
                               preferred_element_type=jnp.float32)
    return pl.pallas_call(
        body,
        grid=(NP // _BLK,),
        in_specs=[pl.BlockSpec((_BLK, D), lambda i: (i, 0)),
                  pl.BlockSpec((_BLK, D), lambda i: (i, 0)),
                  pl.BlockSpec((_BLK, 16), lambda i: (i, 0)),
                  pl.BlockSpec((_BLK, 16), lambda i: (i, 0)),
                  pl.BlockSpec((D, 3 * D), lambda i: (0, 0))],
        out_specs=[pl.BlockSpec((_BLK, D), lambda i: (i, 0)),
                   pl.BlockSpec((_BLK, 3 * D), lambda i: (i, 0))],
        out_shape=[jax.ShapeDtypeStruct((NP, D), jnp.float32),
                   jax.ShapeDtypeStruct((NP, 3 * D), jnp.float32)],
    )(a0, a1, n0, n1, w)


def _tc_qkv0(a0, a1, w):
    def body(a0_ref, a1_ref, w_ref, comb_ref, qkv_ref):
        a = a0_ref[...] + a1_ref[...]
        comb_ref[...] = a
        qkv_ref[...] = jnp.dot(a, w_ref[...],
                               preferred_element_type=jnp.float32)
    return pl.pallas_call(
        body,
        grid=(NP // _BLK,),
        in_specs=[pl.BlockSpec((_BLK, D), lambda i: (i, 0)),
                  pl.BlockSpec((_BLK, D), lambda i: (i, 0)),
                  pl.BlockSpec((D, 3 * D), lambda i: (0, 0))],
        out_specs=[pl.BlockSpec((_BLK, D), lambda i: (i, 0)),
                   pl.BlockSpec((_BLK, 3 * D), lambda i: (i, 0))],
        out_shape=[jax.ShapeDtypeStruct((NP, D), jnp.float32),
                   jax.ShapeDtypeStruct((NP, 3 * D), jnp.float32)],
    )(a0, a1, w)


def _tc_sum2(a0, a1):
    def body(a0_ref, a1_ref, o_ref):
        o_ref[...] = a0_ref[...] + a1_ref[...]
    return pl.pallas_call(
        body,
        grid=(NP // _BLK,),
        in_specs=[pl.BlockSpec((_BLK, D), lambda i: (i, 0))] * 2,
        out_specs=pl.BlockSpec((_BLK, D), lambda i: (i, 0)),
        out_shape=jax.ShapeDtypeStruct((NP, D), jnp.float32),
    )(a0, a1)


def _tc_final(e0, e1, e2, e3, t0, t1, n0, n1):
    def body(e0_ref, e1_ref, e2_ref, e3_ref, t0_ref, t1_ref, n0_ref, n1_ref,
             o_ref):
        e4 = _norm_div(t0_ref, t1_ref, n0_ref, n1_ref)
        o_ref[...] = (e0_ref[...] + e1_ref[...] + e2_ref[...]
                      + e3_ref[...] + e4)
    return pl.pallas_call(
        body,
        grid=(NP // _BLK,),
        in_specs=[pl.BlockSpec((_BLK, D), lambda i: (i, 0))] * 6
                 + [pl.BlockSpec((_BLK, 16), lambda i: (i, 0))] * 2,
        out_specs=pl.BlockSpec((_BLK, D), lambda i: (i, 0)),
        out_shape=jax.ShapeDtypeStruct((NP, D), jnp.float32),
    )(e0, e1, e2, e3, t0, t1, n0, n1)


def kernel(enc_edge_index, enc_values, dec_edge_index, uEmbeds, iEmbeds,
           qTrans0, kTrans0, vTrans0, filter0,
           qTrans1, kTrans1, vTrans1, filter1):
    f32 = jnp.float32
    e0 = jnp.concatenate(
        [uEmbeds, iEmbeds, jnp.zeros((NP - N, D), f32)], axis=0)
    zeros_nd = jnp.zeros((NP, D), f32)
    zeros_n16 = jnp.zeros((NP, 16), f32)

    pad = EP - E
    enc_rows = jnp.concatenate(
        [enc_edge_index[0], jnp.full((pad,), N, jnp.int32)])
    enc_cols = jnp.concatenate(
        [enc_edge_index[1], jnp.full((pad,), N, jnp.int32)])
    enc_vals = jnp.concatenate(
        [enc_values, jnp.zeros((pad,), f32)]).reshape(EP // 16, 16)
    dec_rows = jnp.concatenate(
        [dec_edge_index[0], jnp.full((pad,), N, jnp.int32)])
    dec_cols = jnp.concatenate(
        [dec_edge_index[1], jnp.full((pad,), N, jnp.int32)])

    f0p = jnp.pad(jnp.repeat(filter0, 4, axis=1), ((0, NP - N), (0, D - 16)))
    f1p = jnp.pad(jnp.repeat(filter1, 4, axis=1), ((0, NP - N), (0, D - 16)))
    w0 = jnp.concatenate([qTrans0, kTrans0, vTrans0], axis=1)
    w1 = jnp.concatenate([qTrans1, kTrans1, vTrans1], axis=1)

    g1 = _sc_gcn(enc_rows, enc_cols, enc_vals, e0, zeros_nd)
    e1 = _tc_sum2(g1[0], g1[1])
    g2 = _sc_gcn(enc_rows, enc_cols, enc_vals, e1, zeros_nd)

    e2, qkv1 = _tc_qkv0(g2[0], g2[1], w0)
    ea1, np1 = _sc_gt_p1(dec_rows, dec_cols, qkv1[:, :D], qkv1[:, D:2 * D],
                         f0p, zeros_nd)
    t1 = _sc_gt_p2(dec_rows, dec_cols, qkv1[:, 2 * D:], ea1, zeros_nd)

    e3, qkv2 = _tc_qkv(t1[0], t1[1], np1[0][:, :16], np1[1][:, :16], w1)
    ea2, np2 = _sc_gt_p1(dec_rows, dec_cols, qkv2[:, :D], qkv2[:, D:2 * D],
                         f1p, zeros_nd)
    t2 = _sc_gt_p2(dec_rows, dec_cols, qkv2[:, 2 * D:], ea2, zeros_nd)

    out = _tc_final(e0, e1, e2, e3, t2[0], t2[1],
                    np2[0][:, :16], np2[1][:, :16])
    return (out[:USER], out[USER:N])


# speedup vs baseline: 1.1803x; 1.0881x over previous
import functools

import jax
import jax.numpy as jnp
from jax import lax
from jax.experimental import pallas as pl
from jax.experimental.pallas import tpu as pltpu
from jax.experimental.pallas import tpu_sc as plsc

USER = 5000
ITEM = 5000
N = USER + ITEM
E = 320000
D = 128
H = 4
DH = D // H

NW = 32
CE = 128
NP = N + 112
EP = 2560 * CE
CHUNKS_PER_W = (EP // CE) // NW
RPT = NP // 16

_MESH = plsc.VectorSubcoreMesh(core_axis_name="c", subcore_axis_name="s")

_GDN = lax.GatherDimensionNumbers(
    offset_dims=(), collapsed_slice_dims=(0,), start_index_map=(0,))


def _shuf(v, idx):
    return lax.gather(v, idx[:, None], _GDN, (1,),
                      mode=lax.GatherScatterMode.PROMISE_IN_BOUNDS)


def _head_sums(t, iot):
    t = [v + _shuf(v, iot ^ 8) for v in t]
    t = [v + _shuf(v, iot ^ 4) for v in t]
    m = jnp.where(iot < 4, t[0],
        jnp.where(iot < 8, t[1],
        jnp.where(iot < 12, t[2], t[3])))
    m = m + _shuf(m, iot ^ 2)
    m = m + _shuf(m, iot ^ 1)
    return m


@functools.partial(
    pl.kernel,
    out_type=jax.ShapeDtypeStruct((2, NP, D), jnp.float32),
    mesh=_MESH,
    scratch_types=[
        pltpu.VMEM((CE,), jnp.int32),
        pltpu.VMEM((CE,), jnp.int32),
        pltpu.VMEM((CE // 16, 16), jnp.float32),
        pltpu.VMEM((CE, D), jnp.float32),
        pltpu.VMEM_SHARED((NP, D), jnp.float32),
        pltpu.SemaphoreType.DMA,
    ],
)
def _sc_gcn(rows_hbm, cols_hbm, vals_hbm, table_hbm, zeros_hbm, out_hbm,
            rows_buf, cols_buf, vals_buf, ebuf, acc_sp, sem):
    cid = lax.axis_index("c")
    sid = lax.axis_index("s")
    wid = sid * 2 + cid
    pltpu.sync_copy(zeros_hbm.at[pl.ds(sid * RPT, RPT), :],
                    acc_sp.at[pl.ds(sid * RPT, RPT), :])
    plsc.subcore_barrier()

    def body(i, carry):
        cidx = wid + i * NW
        base = cidx * CE
        pltpu.sync_copy(rows_hbm.at[pl.ds(base, CE)], rows_buf)
        pltpu.sync_copy(cols_hbm.at[pl.ds(base, CE)], cols_buf)
        pltpu.sync_copy(vals_hbm.at[pl.ds(cidx * (CE // 16), CE // 16), :],
                        vals_buf)
        pltpu.async_copy(table_hbm.at[cols_buf], ebuf, sem).wait()

        @plsc.parallel_loop(0, CE // 16)
        def grp_body(g):
            vals = vals_buf[g, :]
            for j in range(16):
                e = g * 16 + j
                sc = vals[j]
                for q in range(D // 16):
                    ebuf[e, pl.ds(q * 16, 16)] = \
                        ebuf[e, pl.ds(q * 16, 16)] * sc

        pltpu.sync_copy(ebuf, acc_sp.at[rows_buf], add=True)
        return carry

    lax.fori_loop(0, CHUNKS_PER_W, body, 0)
    plsc.subcore_barrier()
    pltpu.sync_copy(acc_sp.at[pl.ds(sid * RPT, RPT), :],
                    out_hbm.at[cid, pl.ds(sid * RPT, RPT), :])


CE1 = 64
CHUNKS_P1 = (EP // CE1) // NW


@functools.partial(
    pl.kernel,
    out_type=(jax.ShapeDtypeStruct((EP // 8, D), jnp.float32),
              jax.ShapeDtypeStruct((2, NP, D), jnp.float32)),
    mesh=_MESH,
    scratch_types=[
        pltpu.VMEM((CE1,), jnp.int32),
        pltpu.VMEM((CE1,), jnp.int32),
        pltpu.VMEM((CE1, D), jnp.float32),
        pltpu.VMEM((CE1, D), jnp.float32),
        pltpu.VMEM((CE1, D), jnp.float32),
        pltpu.VMEM((CE1 // 8, D), jnp.float32),
        pltpu.VMEM_SHARED((NP, D), jnp.float32),
        pltpu.SemaphoreType.DMA,
    ],
)
def _sc_gt_p1(rows_hbm, cols_hbm, q_hbm, k_hbm, f_hbm, zeros_hbm,
              ea_hbm, normp_hbm,
              rows_buf, cols_buf, qbuf, kbuf, fbuf, eabuf, norm_sp, sem):
    cid = lax.axis_index("c")
    sid = lax.axis_index("s")
    wid = sid * 2 + cid
    pltpu.sync_copy(zeros_hbm.at[pl.ds(sid * RPT, RPT), :],
                    norm_sp.at[pl.ds(sid * RPT, RPT), :])
    plsc.subcore_barrier()
    iot = lax.iota(jnp.int32, 16)

    def body(i, carry):
        cidx = wid + i * NW
        base = cidx * CE1
        pltpu.sync_copy(rows_hbm.at[pl.ds(base, CE1)], rows_buf)
        pltpu.sync_copy(cols_hbm.at[pl.ds(base, CE1)], cols_buf)
        cq = pltpu.async_copy(q_hbm.at[rows_buf], qbuf, sem)
        ck = pltpu.async_copy(k_hbm.at[cols_buf], kbuf, sem)
        cf = pltpu.async_copy(f_hbm.at[cols_buf], fbuf, sem)
        cq.wait()
        ck.wait()
        cf.wait()

        @plsc.parallel_loop(0, CE1 // 8)
        def row_body(r):
            for j in range(8):
                e = r * 8 + j
                pr = [qbuf[e, pl.ds(q * 16, 16)] * kbuf[e, pl.ds(q * 16, 16)]
                      for q in range(D // 16)]
                att = _head_sums(
                    [pr[2 * h] + pr[2 * h + 1] for h in range(H)], iot)
                att = jnp.clip(att, -10.0, 10.0) + fbuf[e, pl.ds(0, 16)]
                ex = jnp.exp(att)
                fbuf[e, pl.ds(0, 16)] = ex
                eabuf[r, pl.ds(j * 16, 16)] = ex

        pltpu.sync_copy(eabuf,
                        ea_hbm.at[pl.ds(cidx * (CE1 // 8), CE1 // 8), :])
        pltpu.sync_copy(fbuf, norm_sp.at[rows_buf], add=True)
        return carry

    lax.fori_loop(0, CHUNKS_P1, body, 0)
    plsc.subcore_barrier()
    pltpu.sync_copy(norm_sp.at[pl.ds(sid * RPT, RPT), :],
                    normp_hbm.at[cid, pl.ds(sid * RPT, RPT), :])


@functools.partial(
    pl.kernel,
    out_type=jax.ShapeDtypeStruct((2, NP, D), jnp.float32),
    mesh=_MESH,
    scratch_types=[
        pltpu.VMEM((CE,), jnp.int32),
        pltpu.VMEM((CE,), jnp.int32),
        pltpu.VMEM((CE, D), jnp.float32),
        pltpu.VMEM((CE // 8, D), jnp.float32),
        pltpu.VMEM_SHARED((NP, D), jnp.float32),
        pltpu.SemaphoreType.DMA,
    ],
)
def _sc_gt_p2(rows_hbm, cols_hbm, v_hbm, ea_hbm, zeros_hbm, out_hbm,
              rows_buf, cols_buf, vbuf, abuf, acc_sp, sem):
    cid = lax.axis_index("c")
    sid = lax.axis_index("s")
    wid = sid * 2 + cid
    pltpu.sync_copy(zeros_hbm.at[pl.ds(sid * RPT, RPT), :],
                    acc_sp.at[pl.ds(sid * RPT, RPT), :])
    plsc.subcore_barrier()

    def body(i, carry):
        cidx = wid + i * NW
        base = cidx * CE
        pltpu.sync_copy(rows_hbm.at[pl.ds(base, CE)], rows_buf)
        pltpu.sync_copy(cols_hbm.at[pl.ds(base, CE)], cols_buf)
        cv = pltpu.async_copy(v_hbm.at[cols_buf], vbuf, sem)
        pltpu.sync_copy(ea_hbm.at[pl.ds(cidx * (CE // 8), CE // 8), :], abuf)
        cv.wait()

        @plsc.parallel_loop(0, CE // 8)
        def row_body(r):
            for j in range(8):
                e = r * 8 + j
                a = abuf[r, pl.ds(j * 16, 16)]
                for h in range(H):
                    sh = a[4 * h]
                    vbuf[e, pl.ds(2 * h * 16, 16)] = \
                        vbuf[e, pl.ds(2 * h * 16, 16)] * sh
                    vbuf[e, pl.ds((2 * h + 1) * 16, 16)] = \
                        vbuf[e, pl.ds((2 * h + 1) * 16, 16)] * sh

        pltpu.sync_copy(vbuf, acc_sp.at[rows_buf], add=True)
        return carry

    lax.fori_loop(0, CHUNKS_PER_W, body, 0)
    plsc.subcore_barrier()
    pltpu.sync_copy(acc_sp.at[pl.ds(sid * RPT, RPT), :],
                    out_hbm.at[cid, pl.ds(sid * RPT, RPT), :])


_BLK = NP // 8


def _norm_div(t0_ref, t1_ref, n0_ref, n1_ref):
    n = n0_ref[...] + n1_ref[...] + 1e-8
    n = jnp.repeat(n, DH // 4, axis=1)
    return (t0_ref[...] + t1_ref[...]) / n


def _tc_qkv(a0, a1, n0, n1, w):
    def body(a0_ref, a1_ref, n0_ref, n1_ref, w_ref, comb_ref, qkv_ref):
        a = _norm_div(a0_ref, a1_ref, n0_ref, n1_ref)
        comb_ref[...] = a
        qkv_ref[...] = jnp.dot(a, w_ref[...],
                               preferred_element_type=jnp.float32)
    return pl.pallas_call(
        body,
        grid=(NP // _BLK,),
        in_specs=[pl.BlockSpec((_BLK, D), lambda i: (i, 0)),
                  pl.BlockSpec((_BLK, D), lambda i: (i, 0)),
                  pl.BlockSpec((_BLK, 16), lambda i: (i, 0)),
                  pl.BlockSpec((_BLK, 16), lambda i: (i, 0)),
                  pl.BlockSpec((D, 3 * D), lambda i: (0, 0))],
        out_specs=[pl.BlockSpec((_BLK, D), lambda i: (i, 0)),
                   pl.BlockSpec((_BLK, 3 * D), lambda i: (i, 0))],
        out_shape=[jax.ShapeDtypeStruct((NP, D), jnp.float32),
                   jax.ShapeDtypeStruct((NP, 3 * D), jnp.float32)],
    )(a0, a1, n0, n1, w)


def _tc_qkv0(a0, a1, w):
    def body(a0_ref, a1_ref, w_ref, comb_ref, qkv_ref):
        a = a0_ref[...] + a1_ref[...]
        comb_ref[...] = a
        qkv_ref[...] = jnp.dot(a, w_ref[...],
                               preferred_element_type=jnp.float32)
    return pl.pallas_call(
        body,
        grid=(NP // _BLK,),
        in_specs=[pl.BlockSpec((_BLK, D), lambda i: (i, 0)),
                  pl.BlockSpec((_BLK, D), lambda i: (i, 0)),
                  pl.BlockSpec((D, 3 * D), lambda i: (0, 0))],
        out_specs=[pl.BlockSpec((_BLK, D), lambda i: (i, 0)),
                   pl.BlockSpec((_BLK, 3 * D), lambda i: (i, 0))],
        out_shape=[jax.ShapeDtypeStruct((NP, D), jnp.float32),
                   jax.ShapeDtypeStruct((NP, 3 * D), jnp.float32)],
    )(a0, a1, w)


def _tc_sum2(a0, a1):
    def body(a0_ref, a1_ref, o_ref):
        o_ref[...] = a0_ref[...] + a1_ref[...]
    return pl.pallas_call(
        body,
        grid=(NP // _BLK,),
        in_specs=[pl.BlockSpec((_BLK, D), lambda i: (i, 0))] * 2,
        out_specs=pl.BlockSpec((_BLK, D), lambda i: (i, 0)),
        out_shape=jax.ShapeDtypeStruct((NP, D), jnp.float32),
    )(a0, a1)


def _tc_final(e0, e1, e2, e3, t0, t1, n0, n1):
    def body(e0_ref, e1_ref, e2_ref, e3_ref, t0_ref, t1_ref, n0_ref, n1_ref,
             o_ref):
        e4 = _norm_div(t0_ref, t1_ref, n0_ref, n1_ref)
        o_ref[...] = (e0_ref[...] + e1_ref[...] + e2_ref[...]
                      + e3_ref[...] + e4)
    return pl.pallas_call(
        body,
        grid=(NP // _BLK,),
        in_specs=[pl.BlockSpec((_BLK, D), lambda i: (i, 0))] * 6
                 + [pl.BlockSpec((_BLK, 16), lambda i: (i, 0))] * 2,
        out_specs=pl.BlockSpec((_BLK, D), lambda i: (i, 0)),
        out_shape=jax.ShapeDtypeStruct((NP, D), jnp.float32),
    )(e0, e1, e2, e3, t0, t1, n0, n1)


def kernel(enc_edge_index, enc_values, dec_edge_index, uEmbeds, iEmbeds,
           qTrans0, kTrans0, vTrans0, filter0,
           qTrans1, kTrans1, vTrans1, filter1):
    f32 = jnp.float32
    e0 = jnp.concatenate(
        [uEmbeds, iEmbeds, jnp.zeros((NP - N, D), f32)], axis=0)
    zeros_nd = jnp.zeros((NP, D), f32)
    zeros_n16 = jnp.zeros((NP, 16), f32)

    pad = EP - E
    enc_rows = jnp.concatenate(
        [enc_edge_index[0], jnp.full((pad,), N, jnp.int32)])
    enc_cols = jnp.concatenate(
        [enc_edge_index[1], jnp.full((pad,), N, jnp.int32)])
    enc_vals = jnp.concatenate(
        [enc_values, jnp.zeros((pad,), f32)]).reshape(EP // 16, 16)
    dec_rows = jnp.concatenate(
        [dec_edge_index[0], jnp.full((pad,), N, jnp.int32)])
    dec_cols = jnp.concatenate(
        [dec_edge_index[1], jnp.full((pad,), N, jnp.int32)])

    f0p = jnp.pad(jnp.repeat(filter0, 4, axis=1), ((0, NP - N), (0, D - 16)))
    f1p = jnp.pad(jnp.repeat(filter1, 4, axis=1), ((0, NP - N), (0, D - 16)))
    w0 = jnp.concatenate([qTrans0, kTrans0, vTrans0], axis=1)
    w1 = jnp.concatenate([qTrans1, kTrans1, vTrans1], axis=1)

    g1 = _sc_gcn(enc_rows, enc_cols, enc_vals, e0, zeros_nd)
    e1 = _tc_sum2(g1[0], g1[1])
    g2 = _sc_gcn(enc_rows, enc_cols, enc_vals, e1, zeros_nd)

    e2, qkv1 = _tc_qkv0(g2[0], g2[1], w0)
    ea1, np1 = _sc_gt_p1(dec_rows, dec_cols, qkv1[:, :D], qkv1[:, D:2 * D],
                         f0p, zeros_nd)
    t1 = _sc_gt_p2(dec_rows, dec_cols, qkv1[:, 2 * D:], ea1, zeros_nd)

    e3, qkv2 = _tc_qkv(t1[0], t1[1], np1[0][:, :16], np1[1][:, :16], w1)
    ea2, np2 = _sc_gt_p1(dec_rows, dec_cols, qkv2[:, :D], qkv2[:, D:2 * D],
                         f1p, zeros_nd)
    t2 = _sc_gt_p2(dec_rows, dec_cols, qkv2[:, 2 * D:], ea2, zeros_nd)

    out = _tc_final(e0, e1, e2, e3, t2[0], t2[1],
                    np2[0][:, :16], np2[1][:, :16])
    return (out[:USER], out[USER:N])
